# Initial kernel scaffold; baseline (speedup 1.0000x reference)
import jax
import jax.numpy as jnp
from jax.experimental import pallas as pl


def kernel(in_feat, segment_ids, weight, bias):
    raise NotImplementedError("write your pallas kernel here")


# speedup vs baseline: 4.0367x; 4.0367x over previous
import jax
import jax.numpy as jnp
from jax import lax
from jax.experimental import pallas as pl
from jax.experimental.pallas import tpu as pltpu

NSEG = 256
BLK = 512


def _stats_body(x_ref, ids_ref, sum_ref, sq_ref, cnt_ref):
    i = pl.program_id(0)

    @pl.when(i == 0)
    def _():
        sum_ref[...] = jnp.zeros_like(sum_ref)
        sq_ref[...] = jnp.zeros_like(sq_ref)
        cnt_ref[...] = jnp.zeros_like(cnt_ref)

    ids = ids_ref[0]
    seg = lax.broadcasted_iota(jnp.int32, (NSEG, BLK), 0)
    onehot_t = (seg == ids).astype(jnp.float32)
    x = x_ref[...]
    sum_ref[...] += jnp.dot(onehot_t, x, preferred_element_type=jnp.float32)
    sq_ref[...] += jnp.dot(onehot_t, x * x, preferred_element_type=jnp.float32)
    cnt_ref[...] += jnp.sum(onehot_t, axis=1, keepdims=True)


def _norm_body(x_ref, ids_ref, sum_ref, sq_ref, cnt_ref, w_ref, b_ref,
               o_ref, scale_ref, shift_ref):
    i = pl.program_id(0)

    @pl.when(i == 0)
    def _():
        cnt = jnp.maximum(cnt_ref[:, :1], 1.0)
        mean = sum_ref[...] / cnt
        var = sq_ref[...] / cnt - mean * mean
        inv = lax.rsqrt(var + 1e-8)
        w = w_ref[...]
        scale_ref[...] = inv * w
        shift_ref[...] = b_ref[...] - mean * inv * w

    ids = ids_ref[0]
    seg = lax.broadcasted_iota(jnp.int32, (NSEG, BLK), 0)
    onehot_t = (seg == ids).astype(jnp.float32)
    dn = (((0,), (0,)), ((), ()))
    srow = lax.dot_general(onehot_t, scale_ref[...], dn,
                           preferred_element_type=jnp.float32)
    trow = lax.dot_general(onehot_t, shift_ref[...], dn,
                           preferred_element_type=jnp.float32)
    o_ref[...] = x_ref[...] * srow + trow


def kernel(in_feat, segment_ids, weight, bias):
    n, d = in_feat.shape
    nblk = n // BLK
    ids = segment_ids.astype(jnp.int32).reshape(nblk, 1, BLK)

    sums, sq, cnt = pl.pallas_call(
        _stats_body,
        grid=(nblk,),
        in_specs=[
            pl.BlockSpec((BLK, d), lambda i: (i, 0)),
            pl.BlockSpec((1, 1, BLK), lambda i: (i, 0, 0)),
        ],
        out_specs=[
            pl.BlockSpec((NSEG, d), lambda i: (0, 0)),
            pl.BlockSpec((NSEG, d), lambda i: (0, 0)),
            pl.BlockSpec((NSEG, d), lambda i: (0, 0)),
        ],
        out_shape=[
            jax.ShapeDtypeStruct((NSEG, d), jnp.float32),
            jax.ShapeDtypeStruct((NSEG, d), jnp.float32),
            jax.ShapeDtypeStruct((NSEG, d), jnp.float32),
        ],
    )(in_feat, ids)

    out = pl.pallas_call(
        _norm_body,
        grid=(nblk,),
        in_specs=[
            pl.BlockSpec((BLK, d), lambda i: (i, 0)),
            pl.BlockSpec((1, 1, BLK), lambda i: (i, 0, 0)),
            pl.BlockSpec((NSEG, d), lambda i: (0, 0)),
            pl.BlockSpec((NSEG, d), lambda i: (0, 0)),
            pl.BlockSpec((NSEG, d), lambda i: (0, 0)),
            pl.BlockSpec((1, d), lambda i: (0, 0)),
            pl.BlockSpec((1, d), lambda i: (0, 0)),
        ],
        out_specs=pl.BlockSpec((BLK, d), lambda i: (i, 0)),
        out_shape=jax.ShapeDtypeStruct((n, d), jnp.float32),
        scratch_shapes=[
            pltpu.VMEM((NSEG, d), jnp.float32),
            pltpu.VMEM((NSEG, d), jnp.float32),
        ],
    )(in_feat, ids, sums, sq, cnt, weight, bias)
    return out


# speedup vs baseline: 6.6768x; 1.6540x over previous
import jax
import jax.numpy as jnp
from jax import lax
from jax.experimental import pallas as pl
from jax.experimental.pallas import tpu as pltpu

NSEG = 256
BLK = 2560


def _stats_body(x_ref, ids_ref, sum_ref, sq_ref, cnt_ref):
    i = pl.program_id(0)

    @pl.when(i == 0)
    def _():
        sum_ref[...] = jnp.zeros_like(sum_ref)
        sq_ref[...] = jnp.zeros_like(sq_ref)
        cnt_ref[...] = jnp.zeros_like(cnt_ref)

    ids = ids_ref[0]
    first = jnp.min(ids)
    last = jnp.max(ids)
    x = x_ref[...]
    xx = x * x
    riota = lax.broadcasted_iota(jnp.int32, (BLK, 1), 0)

    def body(s, carry):
        lo = jnp.sum((ids < s).astype(jnp.int32))
        hi = jnp.sum((ids <= s).astype(jnp.int32))
        m = (riota >= lo) & (riota < hi)
        sum_ref[pl.ds(s, 1), :] += jnp.sum(
            jnp.where(m, x, 0.0), axis=0, keepdims=True)
        sq_ref[pl.ds(s, 1), :] += jnp.sum(
            jnp.where(m, xx, 0.0), axis=0, keepdims=True)
        cnt_ref[pl.ds(s, 1), :] += jnp.full(
            (1, x.shape[1]), 1.0) * (hi - lo).astype(jnp.float32)
        return carry

    lax.fori_loop(first, last + 1, body, 0)


def _norm_body(x_ref, ids_ref, sum_ref, sq_ref, cnt_ref, w_ref, b_ref,
               o_ref, scale_ref, shift_ref):
    i = pl.program_id(0)

    @pl.when(i == 0)
    def _():
        cnt = jnp.maximum(cnt_ref[:, :1], 1.0)
        mean = sum_ref[...] / cnt
        var = sq_ref[...] / cnt - mean * mean
        inv = lax.rsqrt(var + 1e-8)
        w = w_ref[...]
        scale_ref[...] = inv * w
        shift_ref[...] = b_ref[...] - mean * inv * w

    ids = ids_ref[0]
    first = jnp.min(ids)
    last = jnp.max(ids)
    x = x_ref[...]
    riota = lax.broadcasted_iota(jnp.int32, (BLK, 1), 0)

    acc = x * scale_ref[pl.ds(first, 1), :] + shift_ref[pl.ds(first, 1), :]

    def body(s, acc):
        lo = jnp.sum((ids < s).astype(jnp.int32))
        m = riota >= lo
        val = x * scale_ref[pl.ds(s, 1), :] + shift_ref[pl.ds(s, 1), :]
        return jnp.where(m, val, acc)

    o_ref[...] = lax.fori_loop(first + 1, last + 1, body, acc)


def kernel(in_feat, segment_ids, weight, bias):
    n, d = in_feat.shape
    nblk = n // BLK
    ids = segment_ids.astype(jnp.int32).reshape(nblk, 1, BLK)

    sums, sq, cnt = pl.pallas_call(
        _stats_body,
        grid=(nblk,),
        in_specs=[
            pl.BlockSpec((BLK, d), lambda i: (i, 0)),
            pl.BlockSpec((1, 1, BLK), lambda i: (i, 0, 0)),
        ],
        out_specs=[
            pl.BlockSpec((NSEG, d), lambda i: (0, 0)),
            pl.BlockSpec((NSEG, d), lambda i: (0, 0)),
            pl.BlockSpec((NSEG, d), lambda i: (0, 0)),
        ],
        out_shape=[
            jax.ShapeDtypeStruct((NSEG, d), jnp.float32),
            jax.ShapeDtypeStruct((NSEG, d), jnp.float32),
            jax.ShapeDtypeStruct((NSEG, d), jnp.float32),
        ],
    )(in_feat, ids)

    out = pl.pallas_call(
        _norm_body,
        grid=(nblk,),
        in_specs=[
            pl.BlockSpec((BLK, d), lambda i: (i, 0)),
            pl.BlockSpec((1, 1, BLK), lambda i: (i, 0, 0)),
            pl.BlockSpec((NSEG, d), lambda i: (0, 0)),
            pl.BlockSpec((NSEG, d), lambda i: (0, 0)),
            pl.BlockSpec((NSEG, d), lambda i: (0, 0)),
            pl.BlockSpec((1, d), lambda i: (0, 0)),
            pl.BlockSpec((1, d), lambda i: (0, 0)),
        ],
        out_specs=pl.BlockSpec((BLK, d), lambda i: (i, 0)),
        out_shape=jax.ShapeDtypeStruct((n, d), jnp.float32),
        scratch_shapes=[
            pltpu.VMEM((NSEG, d), jnp.float32),
            pltpu.VMEM((NSEG, d), jnp.float32),
        ],
    )(in_feat, ids, sums, sq, cnt, weight, bias)
    return out
